# baseline (speedup 1.0000x reference)
import functools

import jax
import jax.numpy as jnp
from jax import lax
from jax.experimental import pallas as pl
from jax.experimental.pallas import tpu as pltpu
import jax.experimental.pallas.tpu_sc as plsc

N = 10000
E = 320000
D = 128
H = 128
OUT = 128

NC = 2
NS = 16
NW = NC * NS
EPW = E // NW
C = 96
NCHUNK = (EPW + C - 1) // C
EPWP = NCHUNK * C
NROW = N + 8
RB = 40
NRB = N // RB
NB = 3


def _sc_segment_sum(W_edge, srcp, dstp):
    mesh = plsc.VectorSubcoreMesh(
        core_axis_name="c", subcore_axis_name="s",
        num_cores=NC, num_subcores=NS)

    @functools.partial(
        pl.kernel,
        mesh=mesh,
        out_type=jax.ShapeDtypeStruct((NC, N, H), jnp.float32),
        scratch_types=[
            pltpu.VMEM((NB, C), jnp.int32),
            pltpu.VMEM((NB, C), jnp.int32),
            pltpu.VMEM((C, H), jnp.float32),
            pltpu.VMEM((C, H), jnp.float32),
            pltpu.VMEM((C, H), jnp.float32),
            pltpu.VMEM_SHARED((NROW, H), jnp.float32),
            pltpu.SemaphoreType.DMA((NB,)),
            pltpu.SemaphoreType.DMA((NB,)),
            pltpu.SemaphoreType.DMA((NB,)),
        ],
    )
    def k(w_hbm, src_hbm, dst_hbm, out_hbm,
          src_v, dst_v, rows0, rows1, rows2, acc, semS, semD, semG):
        cid = lax.axis_index("c")
        sid = lax.axis_index("s")
        wid = sid * NC + cid
        rows = (rows0, rows1, rows2)

        def load_idx(i, b):
            base = wid * EPWP + i * C
            pltpu.async_copy(src_hbm.at[pl.ds(base, C)], src_v.at[b],
                             semS.at[b])
            pltpu.async_copy(dst_hbm.at[pl.ds(base, C)], dst_v.at[b],
                             semD.at[b])

        def wait_idx(i, b):
            base = wid * EPWP + i * C
            pltpu.make_async_copy(src_hbm.at[pl.ds(base, C)], src_v.at[b],
                                  semS.at[b]).wait()
            pltpu.make_async_copy(dst_hbm.at[pl.ds(base, C)], dst_v.at[b],
                                  semD.at[b]).wait()

        def gather(b):
            pltpu.async_copy(w_hbm.at[src_v.at[b]], rows[b], semG.at[b])

        def wait_gather(b):
            pltpu.make_async_copy(w_hbm.at[src_v.at[b]], rows[b],
                                  semG.at[b]).wait()

        def scat(b):
            pltpu.sync_copy(rows[b], acc.at[dst_v.at[b]], add=True)

        for b in range(NB):
            load_idx(b, b)

        def zero_row(r, carry):
            for g in range(H // 16):
                rows0[r, pl.ds(g * 16, 16)] = jnp.zeros((16,), jnp.float32)
            return carry
        lax.fori_loop(0, RB, zero_row, 0)
        for j in range((NRB + NS - 1) // NS):
            g = sid + j * NS
            @pl.when(g < NRB)
            def _():
                pltpu.sync_copy(rows0.at[pl.ds(0, RB)],
                                acc.at[pl.ds(g * RB, RB)])
        wait_idx(0, 0)
        gather(0)
        wait_idx(1, 1)
        gather(1)
        plsc.subcore_barrier()

        def half(i, b):
            wait_gather(b)
            scat(b)
            @pl.when(i + 2 < NCHUNK)
            def _():
                wait_idx(i + 2, (b + 2) % NB)
                gather((b + 2) % NB)
            @pl.when(i + 3 < NCHUNK)
            def _():
                load_idx(i + 3, b)

        def body(p, carry):
            half(3 * p, 0)
            half(3 * p + 1, 1)
            half(3 * p + 2, 2)
            return carry
        lax.fori_loop(0, NCHUNK // 3, body, 0)
        for r in range(NCHUNK - (NCHUNK // 3) * 3):
            half((NCHUNK // 3) * 3 + r, r)
        plsc.subcore_barrier()

        for j in range((NRB + NS - 1) // NS):
            g = sid + j * NS
            @pl.when(g < NRB)
            def _():
                pltpu.sync_copy(acc.at[pl.ds(g * RB, RB)],
                                out_hbm.at[cid, pl.ds(g * RB, RB)])

    return k(W_edge, srcp, dstp)


BN = 2000


def _tc_tail_kernel(s_ref, x_ref, wc1_ref, bc1_ref, wn_ref, bn_ref,
                    wc2_ref, bc2_ref, wf_ref, bf_ref, be_ref, out_ref):
    s = s_ref[0] + s_ref[1] + be_ref[...]
    t = s + jnp.dot(s, wc1_ref[...], preferred_element_type=jnp.float32) \
        + bc1_ref[...]
    h = jnp.dot(x_ref[...], wn_ref[...], preferred_element_type=jnp.float32) \
        + bn_ref[...]
    t = t + h + jnp.dot(h, wc2_ref[...], preferred_element_type=jnp.float32) \
        + bc2_ref[...]
    t = jnp.maximum(t, 0.0)
    out_ref[...] = jnp.dot(t, wf_ref[...],
                           preferred_element_type=jnp.float32) + bf_ref[...]


def _tc_tail(S2, x, W_cat1, b_cat1, W_node, b_node, W_cat2, b_cat2,
             W_final, b_final, b_edge):
    full = lambda shape: pl.BlockSpec(shape, lambda i: (0, 0))
    return pl.pallas_call(
        _tc_tail_kernel,
        grid=(N // BN,),
        in_specs=[
            pl.BlockSpec((NC, BN, H), lambda i: (0, i, 0)),
            pl.BlockSpec((BN, D), lambda i: (i, 0)),
            full((H, H)), full((1, H)),
            full((D, H)), full((1, H)),
            full((H, H)), full((1, H)),
            full((H, OUT)), full((1, OUT)),
            full((1, H)),
        ],
        out_specs=pl.BlockSpec((BN, OUT), lambda i: (i, 0)),
        out_shape=jax.ShapeDtypeStruct((N, OUT), jnp.float32),
    )(S2, x, W_cat1, b_cat1, W_node, b_node, W_cat2, b_cat2,
      W_final, b_final, b_edge)


def kernel(x, edge_index, W_edge, b_edge, W_node, b_node,
           W_cat1, b_cat1, W_cat2, b_cat2, W_final, b_final):
    pad = EPWP - EPW
    srcp = jnp.pad(edge_index[0].reshape(NW, EPW), ((0, 0), (0, pad)),
                   constant_values=0).reshape(NW * EPWP)
    dstp = jnp.pad(edge_index[1].reshape(NW, EPW), ((0, 0), (0, pad)),
                   constant_values=N).reshape(NW * EPWP)
    S2 = _sc_segment_sum(W_edge, srcp, dstp)
    return _tc_tail(S2, x,
                    W_cat1, b_cat1.reshape(1, H),
                    W_node, b_node.reshape(1, H),
                    W_cat2, b_cat2.reshape(1, H),
                    W_final, b_final.reshape(1, OUT),
                    b_edge.reshape(1, H))


# speedup vs baseline: 1.0188x; 1.0188x over previous
import functools

import jax
import jax.numpy as jnp
from jax import lax
from jax.experimental import pallas as pl
from jax.experimental.pallas import tpu as pltpu
import jax.experimental.pallas.tpu_sc as plsc

N = 10000
E = 320000
D = 128
H = 128
OUT = 128

NC = 2
NS = 16
NW = NC * NS
EPW = E // NW
C = 112
NCHUNK = (EPW + C - 1) // C
EPWP = NCHUNK * C
NROW = N + 8
RB = 40
NRB = N // RB
NB = 3


def _sc_segment_sum(W_edge, srcp, dstp):
    mesh = plsc.VectorSubcoreMesh(
        core_axis_name="c", subcore_axis_name="s",
        num_cores=NC, num_subcores=NS)

    @functools.partial(
        pl.kernel,
        mesh=mesh,
        out_type=jax.ShapeDtypeStruct((NC, N, H), jnp.float32),
        scratch_types=[
            pltpu.VMEM((NB, C), jnp.int32),
            pltpu.VMEM((NB, C), jnp.int32),
            pltpu.VMEM((C, H), jnp.float32),
            pltpu.VMEM((C, H), jnp.float32),
            pltpu.VMEM((C, H), jnp.float32),
            pltpu.VMEM_SHARED((NROW, H), jnp.float32),
            pltpu.SemaphoreType.DMA((NB,)),
            pltpu.SemaphoreType.DMA((NB,)),
            pltpu.SemaphoreType.DMA((NB,)),
        ],
    )
    def k(w_hbm, src_hbm, dst_hbm, out_hbm,
          src_v, dst_v, rows0, rows1, rows2, acc, semS, semD, semG):
        cid = lax.axis_index("c")
        sid = lax.axis_index("s")
        wid = sid * NC + cid
        rows = (rows0, rows1, rows2)

        def load_idx(i, b):
            base = wid * EPWP + i * C
            pltpu.async_copy(src_hbm.at[pl.ds(base, C)], src_v.at[b],
                             semS.at[b])
            pltpu.async_copy(dst_hbm.at[pl.ds(base, C)], dst_v.at[b],
                             semD.at[b])

        def wait_idx(i, b):
            base = wid * EPWP + i * C
            pltpu.make_async_copy(src_hbm.at[pl.ds(base, C)], src_v.at[b],
                                  semS.at[b]).wait()
            pltpu.make_async_copy(dst_hbm.at[pl.ds(base, C)], dst_v.at[b],
                                  semD.at[b]).wait()

        def gather(b):
            pltpu.async_copy(w_hbm.at[src_v.at[b]], rows[b], semG.at[b])

        def wait_gather(b):
            pltpu.make_async_copy(w_hbm.at[src_v.at[b]], rows[b],
                                  semG.at[b]).wait()

        def scat(b):
            pltpu.sync_copy(rows[b], acc.at[dst_v.at[b]], add=True)

        for b in range(NB):
            load_idx(b, b)

        def zero_row(r, carry):
            for g in range(H // 16):
                rows0[r, pl.ds(g * 16, 16)] = jnp.zeros((16,), jnp.float32)
            return carry
        lax.fori_loop(0, RB, zero_row, 0)
        for j in range((NRB + NS - 1) // NS):
            g = sid + j * NS
            @pl.when(g < NRB)
            def _():
                pltpu.sync_copy(rows0.at[pl.ds(0, RB)],
                                acc.at[pl.ds(g * RB, RB)])
        wait_idx(0, 0)
        gather(0)
        wait_idx(1, 1)
        gather(1)
        plsc.subcore_barrier()

        def half(i, b):
            wait_gather(b)
            scat(b)
            @pl.when(i + 2 < NCHUNK)
            def _():
                wait_idx(i + 2, (b + 2) % NB)
                gather((b + 2) % NB)
            @pl.when(i + 3 < NCHUNK)
            def _():
                load_idx(i + 3, b)

        def body(p, carry):
            half(3 * p, 0)
            half(3 * p + 1, 1)
            half(3 * p + 2, 2)
            return carry
        lax.fori_loop(0, NCHUNK // 3, body, 0)
        for r in range(NCHUNK - (NCHUNK // 3) * 3):
            half((NCHUNK // 3) * 3 + r, r)
        plsc.subcore_barrier()

        for j in range((NRB + NS - 1) // NS):
            g = sid + j * NS
            @pl.when(g < NRB)
            def _():
                pltpu.sync_copy(acc.at[pl.ds(g * RB, RB)],
                                out_hbm.at[cid, pl.ds(g * RB, RB)])

    return k(W_edge, srcp, dstp)


BN = 2000


def _tc_tail_kernel(s_ref, x_ref, wc1_ref, bc1_ref, wn_ref, bn_ref,
                    wc2_ref, bc2_ref, wf_ref, bf_ref, be_ref, out_ref):
    s = s_ref[0] + s_ref[1] + be_ref[...]
    t = s + jnp.dot(s, wc1_ref[...], preferred_element_type=jnp.float32) \
        + bc1_ref[...]
    h = jnp.dot(x_ref[...], wn_ref[...], preferred_element_type=jnp.float32) \
        + bn_ref[...]
    t = t + h + jnp.dot(h, wc2_ref[...], preferred_element_type=jnp.float32) \
        + bc2_ref[...]
    t = jnp.maximum(t, 0.0)
    out_ref[...] = jnp.dot(t, wf_ref[...],
                           preferred_element_type=jnp.float32) + bf_ref[...]


def _tc_tail(S2, x, W_cat1, b_cat1, W_node, b_node, W_cat2, b_cat2,
             W_final, b_final, b_edge):
    full = lambda shape: pl.BlockSpec(shape, lambda i: (0, 0))
    return pl.pallas_call(
        _tc_tail_kernel,
        grid=(N // BN,),
        in_specs=[
            pl.BlockSpec((NC, BN, H), lambda i: (0, i, 0)),
            pl.BlockSpec((BN, D), lambda i: (i, 0)),
            full((H, H)), full((1, H)),
            full((D, H)), full((1, H)),
            full((H, H)), full((1, H)),
            full((H, OUT)), full((1, OUT)),
            full((1, H)),
        ],
        out_specs=pl.BlockSpec((BN, OUT), lambda i: (i, 0)),
        out_shape=jax.ShapeDtypeStruct((N, OUT), jnp.float32),
    )(S2, x, W_cat1, b_cat1, W_node, b_node, W_cat2, b_cat2,
      W_final, b_final, b_edge)


def kernel(x, edge_index, W_edge, b_edge, W_node, b_node,
           W_cat1, b_cat1, W_cat2, b_cat2, W_final, b_final):
    pad = EPWP - EPW
    srcp = jnp.pad(edge_index[0].reshape(NW, EPW), ((0, 0), (0, pad)),
                   constant_values=0).reshape(NW * EPWP)
    padblk = jnp.broadcast_to(N + (jnp.arange(pad, dtype=jnp.int32) % 8),
                              (NW, pad))
    dstp = jnp.concatenate(
        [edge_index[1].reshape(NW, EPW), padblk], axis=1).reshape(NW * EPWP)
    S2 = _sc_segment_sum(W_edge, srcp, dstp)
    return _tc_tail(S2, x,
                    W_cat1, b_cat1.reshape(1, H),
                    W_node, b_node.reshape(1, H),
                    W_cat2, b_cat2.reshape(1, H),
                    W_final, b_final.reshape(1, OUT),
                    b_edge.reshape(1, H))


# speedup vs baseline: 1.5770x; 1.5479x over previous
import functools

import jax
import jax.numpy as jnp
from jax import lax
from jax.experimental import pallas as pl
from jax.experimental.pallas import tpu as pltpu
import jax.experimental.pallas.tpu_sc as plsc

N = 10000
E = 320000
D = 128
H = 128
OUT = 128

NC = 2
NS = 16
NW = NC * NS
EPW = E // NW
C = 80
NCHUNK = (EPW + C - 1) // C
EPWP = NCHUNK * C
NROW = N + 8
RB = 40
NRB = N // RB
NB = 3


def _sc_segment_sum(W_edge, srcp, dstp):
    mesh = plsc.VectorSubcoreMesh(
        core_axis_name="c", subcore_axis_name="s",
        num_cores=NC, num_subcores=NS)

    @functools.partial(
        pl.kernel,
        mesh=mesh,
        out_type=jax.ShapeDtypeStruct((NC, N, H), jnp.float32),
        scratch_types=[
            pltpu.VMEM((NB, C), jnp.int32),
            pltpu.VMEM((NB, C), jnp.int32),
            pltpu.VMEM((C, H), jnp.float32),
            pltpu.VMEM((C, H), jnp.float32),
            pltpu.VMEM((C, H), jnp.float32),
            pltpu.VMEM_SHARED((NROW, H), jnp.float32),
            pltpu.SemaphoreType.DMA((NB,)),
            pltpu.SemaphoreType.DMA((NB,)),
            pltpu.SemaphoreType.DMA((NB,)),
        ],
    )
    def k(w_hbm, src_hbm, dst_hbm, out_hbm,
          src_v, dst_v, rows0, rows1, rows2, acc, semS, semD, semG):
        cid = lax.axis_index("c")
        sid = lax.axis_index("s")
        wid = sid * NC + cid
        rows = (rows0, rows1, rows2)

        def load_idx(i, b):
            base = wid * EPWP + i * C
            pltpu.async_copy(src_hbm.at[pl.ds(base, C)], src_v.at[b],
                             semS.at[b])
            pltpu.async_copy(dst_hbm.at[pl.ds(base, C)], dst_v.at[b],
                             semD.at[b])

        def wait_idx(i, b):
            base = wid * EPWP + i * C
            pltpu.make_async_copy(src_hbm.at[pl.ds(base, C)], src_v.at[b],
                                  semS.at[b]).wait()
            pltpu.make_async_copy(dst_hbm.at[pl.ds(base, C)], dst_v.at[b],
                                  semD.at[b]).wait()

        def gather(b):
            pltpu.async_copy(w_hbm.at[src_v.at[b]], rows[b], semG.at[b])

        def wait_gather(b):
            pltpu.make_async_copy(w_hbm.at[src_v.at[b]], rows[b],
                                  semG.at[b]).wait()

        def scat(b):
            pltpu.sync_copy(rows[b], acc.at[dst_v.at[b]], add=True)

        for b in range(NB):
            load_idx(b, b)

        def zero_row(r, carry):
            for g in range(H // 16):
                rows0[r, pl.ds(g * 16, 16)] = jnp.zeros((16,), jnp.float32)
            return carry
        lax.fori_loop(0, RB, zero_row, 0)
        for j in range((NRB + NS - 1) // NS):
            g = sid + j * NS
            @pl.when(g < NRB)
            def _():
                pltpu.sync_copy(rows0.at[pl.ds(0, RB)],
                                acc.at[pl.ds(g * RB, RB)])
        wait_idx(0, 0)
        gather(0)
        wait_idx(1, 1)
        gather(1)
        plsc.subcore_barrier()

        def half(i, b):
            wait_gather(b)
            scat(b)
            @pl.when(i + 2 < NCHUNK)
            def _():
                wait_idx(i + 2, (b + 2) % NB)
                gather((b + 2) % NB)
            @pl.when(i + 3 < NCHUNK)
            def _():
                load_idx(i + 3, b)

        def body(p, carry):
            half(3 * p, 0)
            half(3 * p + 1, 1)
            half(3 * p + 2, 2)
            return carry
        lax.fori_loop(0, NCHUNK // 3, body, 0)
        for r in range(NCHUNK - (NCHUNK // 3) * 3):
            half((NCHUNK // 3) * 3 + r, r)
        plsc.subcore_barrier()

        for j in range((NRB + NS - 1) // NS):
            g = sid + j * NS
            @pl.when(g < NRB)
            def _():
                pltpu.sync_copy(acc.at[pl.ds(g * RB, RB)],
                                out_hbm.at[cid, pl.ds(g * RB, RB)])

    return k(W_edge, srcp, dstp)


BN = 2000


def _tc_tail_kernel(s_ref, x_ref, wc1_ref, bc1_ref, wn_ref, bn_ref,
                    wc2_ref, bc2_ref, wf_ref, bf_ref, be_ref, out_ref):
    s = s_ref[0] + s_ref[1] + be_ref[...]
    t = s + jnp.dot(s, wc1_ref[...], preferred_element_type=jnp.float32) \
        + bc1_ref[...]
    h = jnp.dot(x_ref[...], wn_ref[...], preferred_element_type=jnp.float32) \
        + bn_ref[...]
    t = t + h + jnp.dot(h, wc2_ref[...], preferred_element_type=jnp.float32) \
        + bc2_ref[...]
    t = jnp.maximum(t, 0.0)
    out_ref[...] = jnp.dot(t, wf_ref[...],
                           preferred_element_type=jnp.float32) + bf_ref[...]


def _tc_tail(S2, x, W_cat1, b_cat1, W_node, b_node, W_cat2, b_cat2,
             W_final, b_final, b_edge):
    full = lambda shape: pl.BlockSpec(shape, lambda i: (0, 0))
    return pl.pallas_call(
        _tc_tail_kernel,
        grid=(N // BN,),
        in_specs=[
            pl.BlockSpec((NC, BN, H), lambda i: (0, i, 0)),
            pl.BlockSpec((BN, D), lambda i: (i, 0)),
            full((H, H)), full((1, H)),
            full((D, H)), full((1, H)),
            full((H, H)), full((1, H)),
            full((H, OUT)), full((1, OUT)),
            full((1, H)),
        ],
        out_specs=pl.BlockSpec((BN, OUT), lambda i: (i, 0)),
        out_shape=jax.ShapeDtypeStruct((N, OUT), jnp.float32),
    )(S2, x, W_cat1, b_cat1, W_node, b_node, W_cat2, b_cat2,
      W_final, b_final, b_edge)


def kernel(x, edge_index, W_edge, b_edge, W_node, b_node,
           W_cat1, b_cat1, W_cat2, b_cat2, W_final, b_final):
    pad = EPWP - EPW
    srcp = jnp.pad(edge_index[0].reshape(NW, EPW), ((0, 0), (0, pad)),
                   constant_values=0).reshape(NW * EPWP)
    padblk = jnp.broadcast_to(N + (jnp.arange(pad, dtype=jnp.int32) % 8),
                              (NW, pad))
    dstp = jnp.concatenate(
        [edge_index[1].reshape(NW, EPW), padblk], axis=1).reshape(NW * EPWP)
    S2 = _sc_segment_sum(W_edge, srcp, dstp)
    return _tc_tail(S2, x,
                    W_cat1, b_cat1.reshape(1, H),
                    W_node, b_node.reshape(1, H),
                    W_cat2, b_cat2.reshape(1, H),
                    W_final, b_final.reshape(1, OUT),
                    b_edge.reshape(1, H))
